# baseline (speedup 1.0000x reference)
import functools

import jax
import jax.numpy as jnp
import numpy as np
from jax import lax
from jax.experimental import pallas as pl
from jax.experimental.pallas import tpu as pltpu
from jax.experimental.pallas import tpu_sc as plsc

B = 16384
F = 26
FP = 32
D = 16
BF = B * F
BFP = B * FP

NC = 2
NS = 16
NW = NC * NS
PER_W = BF // NW
PER_W4 = BFP // NW
CH4 = 2048
NCHUNK = PER_W4 // CH4


def _sc_gather(ids4, ids_w, fmw, fmv):
    mesh = plsc.VectorSubcoreMesh(core_axis_name="c", subcore_axis_name="s")

    @functools.partial(
        pl.kernel,
        mesh=mesh,
        compiler_params=pltpu.CompilerParams(use_tc_tiling_on_sc=False),
        out_type=(
            jax.ShapeDtypeStruct((BFP, D), jnp.float32),
            jax.ShapeDtypeStruct((BF,), jnp.float32),
        ),
        scratch_types=[
            pltpu.VMEM((PER_W4,), jnp.int32),
            pltpu.VMEM((PER_W,), jnp.int32),
            pltpu.VMEM((CH4, D), jnp.float32),
            pltpu.VMEM((PER_W,), jnp.float32),
            pltpu.SemaphoreType.DMA,
            pltpu.SemaphoreType.DMA,
        ],
    )
    def gk(ids4_hbm, idsw_hbm, fmw_hbm, fmv_hbm, emb_hbm, w_hbm,
           idx4_v, idxw_v, rows_v, w_v, sem, sem_w):
        wid = lax.axis_index("s") * NC + lax.axis_index("c")
        base4 = wid * PER_W4
        base = wid * PER_W
        pltpu.sync_copy(ids4_hbm.at[pl.ds(base4, PER_W4)], idx4_v)
        pltpu.sync_copy(idsw_hbm.at[pl.ds(base, PER_W)], idxw_v)
        wcopy = pltpu.async_copy(fmw_hbm.at[idxw_v], w_v, sem_w)
        for c in range(NCHUNK):
            pltpu.async_copy(
                fmv_hbm.at[idx4_v.at[pl.ds(c * CH4, CH4)]], rows_v, sem
            ).wait()
            pltpu.sync_copy(rows_v, emb_hbm.at[pl.ds(base4 + c * CH4, CH4)])
        wcopy.wait()
        pltpu.sync_copy(w_v, w_hbm.at[pl.ds(base, PER_W)])

    return gk(ids4, ids_w, fmw, fmv)


BLK = 1024
GRID = B // BLK
FD = FP * D


def _tc_mlp(emb4, vals, w, S, T, W0, b0, W1, b1, W2, b2, Wo, bfin):
    def mk(emb_r, vals_r, w_r, S_r, T_r, W0r, b0r, W1r, b1r, W2r, b2r, Wor,
           bfr, out_r):
        vals_b = vals_r[...]
        y_w = jnp.sum(w_r[...] * vals_b, axis=1, keepdims=True)
        vr = jnp.dot(vals_b, S_r[...], preferred_element_type=jnp.float32,
                     precision=lax.Precision.HIGHEST)
        blk = emb_r[...]
        parts = [jnp.reshape(blk[:, j], (BLK, 128)) for j in range(4)]
        x = jnp.concatenate(parts, axis=1) * vr
        sv = jnp.dot(x, T_r[...], preferred_element_type=jnp.float32,
                     precision=lax.Precision.HIGHEST)
        y_v = 0.5 * (jnp.sum(sv * sv, axis=1, keepdims=True)
                     - jnp.sum(x * x, axis=1, keepdims=True))
        xb = x.astype(jnp.bfloat16)
        h = jnp.maximum(jnp.dot(xb, W0r[...],
                                preferred_element_type=jnp.float32) + b0r[...], 0.0)
        h = jnp.maximum(jnp.dot(h.astype(jnp.bfloat16), W1r[...],
                                preferred_element_type=jnp.float32) + b1r[...], 0.0)
        h = jnp.maximum(jnp.dot(h.astype(jnp.bfloat16), W2r[...],
                                preferred_element_type=jnp.float32) + b2r[...], 0.0)
        y_d = jnp.dot(h, Wor[...], preferred_element_type=jnp.float32)
        y = y_w + y_v + y_d + bfr[...]
        out_r[...] = jax.nn.sigmoid(y)

    full = lambda a: pl.BlockSpec(a.shape, lambda i: (0,) * a.ndim)
    return pl.pallas_call(
        mk,
        grid=(GRID,),
        in_specs=[
            pl.BlockSpec((BLK // 8, 4, 8, 128), lambda i: (i, 0, 0, 0)),
            pl.BlockSpec((BLK, F), lambda i: (i, 0)),
            pl.BlockSpec((BLK, F), lambda i: (i, 0)),
            full(S), full(T), full(W0), full(b0), full(W1), full(b1),
            full(W2), full(b2), full(Wo), full(bfin),
        ],
        out_specs=pl.BlockSpec((BLK, 1), lambda i: (i, 0)),
        out_shape=jax.ShapeDtypeStruct((B, 1), jnp.float32),
    )(emb4, vals, w, S, T, W0, b0, W1, b1, W2, b2, Wo, bfin)


def kernel(feat_ids, feat_vals, FM_B, FM_W, FM_V, params):
    ids32 = jnp.pad(feat_ids.astype(jnp.int32), ((0, 0), (0, FP - F)))
    ids4 = (ids32.reshape(B // 8, 8, 4, 8).transpose(0, 2, 1, 3).reshape(-1))
    ids_w = feat_ids.reshape(-1).astype(jnp.int32)
    emb_flat, w_flat = _sc_gather(ids4, ids_w, FM_W, FM_V)
    emb4 = emb_flat.reshape(B // 8, 4, 8, 128)
    w = w_flat.reshape(B, F)

    a = [params[f"gamma{i}"] * lax.rsqrt(params[f"var{i}"] + 1e-3)
         for i in range(3)]
    c = [params[f"beta{i}"] - params[f"mean{i}"] * a[i] for i in range(3)]
    W0, b0 = params["W0"], params["b0"]
    W1 = a[0][:, None] * params["W1"]
    b1 = c[0] @ params["W1"] + params["b1"]
    W2 = a[1][:, None] * params["W2"]
    b2 = c[1] @ params["W2"] + params["b2"]
    Wo = a[2][:, None] * params["W_out"]
    bfin = c[2] @ params["W_out"] + params["b_out"] + FM_B

    S = np.zeros((F, FD), np.float32)
    S[:, : F * D] = np.repeat(np.eye(F, dtype=np.float32), D, axis=1)
    T = np.zeros((FD, D), np.float32)
    T[: F * D] = np.tile(np.eye(D, dtype=np.float32), (F, 1))
    W0p = jnp.pad(W0, ((0, FD - F * D), (0, 0)))

    pred = _tc_mlp(
        emb4, feat_vals, w, jnp.asarray(S), jnp.asarray(T),
        W0p.astype(jnp.bfloat16), b0.reshape(1, -1),
        W1.astype(jnp.bfloat16), b1.reshape(1, -1),
        W2.astype(jnp.bfloat16), b2.reshape(1, -1), Wo, bfin.reshape(1, 1),
    )
    return pred.reshape(-1)


# speedup vs baseline: 1.5930x; 1.5930x over previous
import functools

import jax
import jax.numpy as jnp
import numpy as np
from jax import lax
from jax.experimental import pallas as pl
from jax.experimental.pallas import tpu as pltpu
from jax.experimental.pallas import tpu_sc as plsc

B = 16384
F = 26
FP = 32
D = 16
BF = B * F
BFP = B * FP

NC = 2
NS = 16
NW = NC * NS
PER_W = BF // NW
PER_W4 = BFP // NW
CH4 = 2048
NCHUNK = PER_W4 // CH4


def _sc_gather(ids4, ids_w, fmw, fmv):
    mesh = plsc.VectorSubcoreMesh(core_axis_name="c", subcore_axis_name="s")

    @functools.partial(
        pl.kernel,
        mesh=mesh,
        compiler_params=pltpu.CompilerParams(use_tc_tiling_on_sc=False),
        out_type=(
            jax.ShapeDtypeStruct((BFP * D // 128, 128), jnp.float32),
            jax.ShapeDtypeStruct((BF,), jnp.float32),
        ),
        scratch_types=[
            pltpu.VMEM((PER_W4,), jnp.int32),
            pltpu.VMEM((PER_W,), jnp.int32),
            pltpu.VMEM((CH4, D), jnp.float32),
            pltpu.VMEM((CH4 * D // 128, 128), jnp.float32),
            pltpu.VMEM((PER_W,), jnp.float32),
            pltpu.SemaphoreType.DMA,
            pltpu.SemaphoreType.DMA,
        ],
    )
    def gk(ids4_hbm, idsw_hbm, fmw_hbm, fmv_hbm, emb_hbm, w_hbm,
           idx4_v, idxw_v, rows_v, out_v, w_v, sem, sem_w):
        wid = lax.axis_index("s") * NC + lax.axis_index("c")
        base4 = wid * PER_W4
        base = wid * PER_W
        orow = wid * (PER_W4 * D // 128)
        crow = CH4 * D // 128
        pltpu.sync_copy(ids4_hbm.at[pl.ds(base4, PER_W4)], idx4_v)
        pltpu.sync_copy(idsw_hbm.at[pl.ds(base, PER_W)], idxw_v)
        wcopy = pltpu.async_copy(fmw_hbm.at[idxw_v], w_v, sem_w)
        for c in range(NCHUNK):
            pltpu.async_copy(
                fmv_hbm.at[idx4_v.at[pl.ds(c * CH4, CH4)]], rows_v, sem
            ).wait()

            def repack(m, _):
                for k in range(8):
                    out_v[m, pl.ds(k * D, D)] = rows_v[m * 8 + k]
                return 0

            lax.fori_loop(0, crow, repack, 0, unroll=2)
            pltpu.sync_copy(out_v, emb_hbm.at[pl.ds(orow + c * crow, crow)])
        wcopy.wait()
        pltpu.sync_copy(w_v, w_hbm.at[pl.ds(base, PER_W)])

    return gk(ids4, ids_w, fmw, fmv)


BLK = 1024
GRID = B // BLK
FD = FP * D


def _tc_mlp(emb4, vals, w, S, T, W0, b0, W1, b1, W2, b2, Wo, bfin):
    def mk(emb_r, vals_r, w_r, S_r, T_r, W0r, b0r, W1r, b1r, W2r, b2r, Wor,
           bfr, out_r):
        vals_b = vals_r[...]
        y_w = jnp.sum(w_r[...] * vals_b, axis=1, keepdims=True)
        vr = jnp.dot(vals_b, S_r[...], preferred_element_type=jnp.float32,
                     precision=lax.Precision.HIGHEST)
        blk = jnp.reshape(emb_r[...], (BLK // 8, 4, 8, 128))
        parts = [jnp.reshape(blk[:, j], (BLK, 128)) for j in range(4)]
        x = jnp.concatenate(parts, axis=1) * vr
        sv = jnp.dot(x, T_r[...], preferred_element_type=jnp.float32,
                     precision=lax.Precision.HIGHEST)
        y_v = 0.5 * (jnp.sum(sv * sv, axis=1, keepdims=True)
                     - jnp.sum(x * x, axis=1, keepdims=True))
        xb = x.astype(jnp.bfloat16)
        h = jnp.maximum(jnp.dot(xb, W0r[...],
                                preferred_element_type=jnp.float32) + b0r[...], 0.0)
        h = jnp.maximum(jnp.dot(h.astype(jnp.bfloat16), W1r[...],
                                preferred_element_type=jnp.float32) + b1r[...], 0.0)
        h = jnp.maximum(jnp.dot(h.astype(jnp.bfloat16), W2r[...],
                                preferred_element_type=jnp.float32) + b2r[...], 0.0)
        y_d = jnp.dot(h, Wor[...], preferred_element_type=jnp.float32)
        y = y_w + y_v + y_d + bfr[...]
        out_r[...] = jax.nn.sigmoid(y)

    full = lambda a: pl.BlockSpec(a.shape, lambda i: (0,) * a.ndim)
    return pl.pallas_call(
        mk,
        grid=(GRID,),
        in_specs=[
            pl.BlockSpec((BLK * FP * D // 128, 128), lambda i: (i, 0)),
            pl.BlockSpec((BLK, F), lambda i: (i, 0)),
            pl.BlockSpec((BLK, F), lambda i: (i, 0)),
            full(S), full(T), full(W0), full(b0), full(W1), full(b1),
            full(W2), full(b2), full(Wo), full(bfin),
        ],
        out_specs=pl.BlockSpec((BLK, 1), lambda i: (i, 0)),
        out_shape=jax.ShapeDtypeStruct((B, 1), jnp.float32),
    )(emb4, vals, w, S, T, W0, b0, W1, b1, W2, b2, Wo, bfin)


def kernel(feat_ids, feat_vals, FM_B, FM_W, FM_V, params):
    pad_ids = jnp.arange(B * (FP - F), dtype=jnp.int32).reshape(B, FP - F)
    ids32 = jnp.concatenate([feat_ids.astype(jnp.int32), pad_ids], axis=1)
    ids4 = (ids32.reshape(B // 8, 8, 4, 8).transpose(0, 2, 1, 3).reshape(-1))
    ids_w = feat_ids.reshape(-1).astype(jnp.int32)
    emb4, w_flat = _sc_gather(ids4, ids_w, FM_W, FM_V)
    w = w_flat.reshape(B, F)

    a = [params[f"gamma{i}"] * lax.rsqrt(params[f"var{i}"] + 1e-3)
         for i in range(3)]
    c = [params[f"beta{i}"] - params[f"mean{i}"] * a[i] for i in range(3)]
    W0, b0 = params["W0"], params["b0"]
    W1 = a[0][:, None] * params["W1"]
    b1 = c[0] @ params["W1"] + params["b1"]
    W2 = a[1][:, None] * params["W2"]
    b2 = c[1] @ params["W2"] + params["b2"]
    Wo = a[2][:, None] * params["W_out"]
    bfin = c[2] @ params["W_out"] + params["b_out"] + FM_B

    S = np.zeros((F, FD), np.float32)
    S[:, : F * D] = np.repeat(np.eye(F, dtype=np.float32), D, axis=1)
    T = np.zeros((FD, D), np.float32)
    T[: F * D] = np.tile(np.eye(D, dtype=np.float32), (F, 1))
    W0p = jnp.pad(W0, ((0, FD - F * D), (0, 0)))

    pred = _tc_mlp(
        emb4, feat_vals, w, jnp.asarray(S), jnp.asarray(T),
        W0p.astype(jnp.bfloat16), b0.reshape(1, -1),
        W1.astype(jnp.bfloat16), b1.reshape(1, -1),
        W2.astype(jnp.bfloat16), b2.reshape(1, -1), Wo, bfin.reshape(1, 1),
    )
    return pred.reshape(-1)


# speedup vs baseline: 1.5934x; 1.0002x over previous
import functools

import jax
import jax.numpy as jnp
import numpy as np
from jax import lax
from jax.experimental import pallas as pl
from jax.experimental.pallas import tpu as pltpu
from jax.experimental.pallas import tpu_sc as plsc

B = 16384
F = 26
FP = 32
D = 16
BF = B * F
BFP = B * FP

NC = 2
NS = 16
NW = NC * NS
PER_W = BF // NW
PER_W4 = BFP // NW
CH4 = 2048
NCHUNK = PER_W4 // CH4


def _sc_gather(ids4, ids_w, fmw, fmv):
    mesh = plsc.VectorSubcoreMesh(core_axis_name="c", subcore_axis_name="s")

    @functools.partial(
        pl.kernel,
        mesh=mesh,
        compiler_params=pltpu.CompilerParams(use_tc_tiling_on_sc=False),
        out_type=(
            jax.ShapeDtypeStruct((BFP * D,), jnp.float32),
            jax.ShapeDtypeStruct((BF,), jnp.float32),
        ),
        scratch_types=[
            pltpu.VMEM((PER_W4,), jnp.int32),
            pltpu.VMEM((PER_W,), jnp.int32),
            pltpu.VMEM((CH4, D), jnp.float32),
            pltpu.VMEM((CH4 * D,), jnp.float32),
            pltpu.VMEM((PER_W,), jnp.float32),
            pltpu.SemaphoreType.DMA,
            pltpu.SemaphoreType.DMA,
        ],
    )
    def gk(ids4_hbm, idsw_hbm, fmw_hbm, fmv_hbm, emb_hbm, w_hbm,
           idx4_v, idxw_v, rows_v, out_v, w_v, sem, sem_w):
        wid = lax.axis_index("s") * NC + lax.axis_index("c")
        base4 = wid * PER_W4
        base = wid * PER_W
        celt = CH4 * D
        pltpu.sync_copy(ids4_hbm.at[pl.ds(base4, PER_W4)], idx4_v)
        pltpu.sync_copy(idsw_hbm.at[pl.ds(base, PER_W)], idxw_v)
        wcopy = pltpu.async_copy(fmw_hbm.at[idxw_v], w_v, sem_w)
        for c in range(NCHUNK):
            pltpu.async_copy(
                fmv_hbm.at[idx4_v.at[pl.ds(c * CH4, CH4)]], rows_v, sem
            ).wait()

            def repack(m, _):
                for k in range(8):
                    out_v[pl.ds(m * 128 + k * D, D)] = rows_v[m * 8 + k]
                return 0

            lax.fori_loop(0, CH4 // 8, repack, 0, unroll=2)
            pltpu.sync_copy(out_v, emb_hbm.at[pl.ds(base4 * D + c * celt, celt)])
        wcopy.wait()
        pltpu.sync_copy(w_v, w_hbm.at[pl.ds(base, PER_W)])

    return gk(ids4, ids_w, fmw, fmv)


BLK = 1024
GRID = B // BLK
FD = FP * D


NB = BLK * FP * D


def _tc_mlp(emb4, vals, w, S, T, W0, b0, W1, b1, W2, b2, Wo, bfin):
    def mk(emb_any, vals_r, w_r, S_r, T_r, W0r, b0r, W1r, b1r, W2r, b2r, Wor,
           bfr, out_r, vbuf, sems):
        i = pl.program_id(0)
        r = lax.rem(i, 2)
        nxt = lax.rem(i + 1, 2)

        @pl.when(i == 0)
        def _():
            pltpu.make_async_copy(
                emb_any.at[pl.ds(0, NB)], vbuf.at[0], sems.at[0]).start()

        @pl.when(i + 1 < GRID)
        def _():
            pltpu.make_async_copy(
                emb_any.at[pl.ds((i + 1) * NB, NB)], vbuf.at[nxt],
                sems.at[nxt]).start()

        pltpu.make_async_copy(
            emb_any.at[pl.ds(i * NB, NB)], vbuf.at[r], sems.at[r]).wait()

        vals_b = vals_r[...]
        y_w = jnp.sum(w_r[...] * vals_b, axis=1, keepdims=True)
        vr = jnp.dot(vals_b, S_r[...], preferred_element_type=jnp.float32,
                     precision=lax.Precision.HIGHEST)
        blk = jnp.reshape(vbuf[r], (BLK // 8, 4, 8, 128))
        parts = [jnp.reshape(blk[:, j], (BLK, 128)) for j in range(4)]
        x = jnp.concatenate(parts, axis=1) * vr
        sv = jnp.dot(x, T_r[...], preferred_element_type=jnp.float32,
                     precision=lax.Precision.HIGHEST)
        y_v = 0.5 * (jnp.sum(sv * sv, axis=1, keepdims=True)
                     - jnp.sum(x * x, axis=1, keepdims=True))
        xb = x.astype(jnp.bfloat16)
        h = jnp.maximum(jnp.dot(xb, W0r[...],
                                preferred_element_type=jnp.float32) + b0r[...], 0.0)
        h = jnp.maximum(jnp.dot(h.astype(jnp.bfloat16), W1r[...],
                                preferred_element_type=jnp.float32) + b1r[...], 0.0)
        h = jnp.maximum(jnp.dot(h.astype(jnp.bfloat16), W2r[...],
                                preferred_element_type=jnp.float32) + b2r[...], 0.0)
        y_d = jnp.dot(h, Wor[...], preferred_element_type=jnp.float32)
        y = y_w + y_v + y_d + bfr[...]
        out_r[...] = jax.nn.sigmoid(y)

    full = lambda a: pl.BlockSpec(a.shape, lambda i: (0,) * a.ndim)
    return pl.pallas_call(
        mk,
        grid=(GRID,),
        in_specs=[
            pl.BlockSpec(memory_space=pl.ANY),
            pl.BlockSpec((BLK, F), lambda i: (i, 0)),
            pl.BlockSpec((BLK, F), lambda i: (i, 0)),
            full(S), full(T), full(W0), full(b0), full(W1), full(b1),
            full(W2), full(b2), full(Wo), full(bfin),
        ],
        out_specs=pl.BlockSpec((BLK, 1), lambda i: (i, 0)),
        out_shape=jax.ShapeDtypeStruct((B, 1), jnp.float32),
        scratch_shapes=[
            pltpu.VMEM((2, NB), jnp.float32),
            pltpu.SemaphoreType.DMA((2,)),
        ],
    )(emb4, vals, w, S, T, W0, b0, W1, b1, W2, b2, Wo, bfin)


def kernel(feat_ids, feat_vals, FM_B, FM_W, FM_V, params):
    pad_ids = jnp.arange(B * (FP - F), dtype=jnp.int32).reshape(B, FP - F)
    ids32 = jnp.concatenate([feat_ids.astype(jnp.int32), pad_ids], axis=1)
    ids4 = (ids32.reshape(B // 8, 8, 4, 8).transpose(0, 2, 1, 3).reshape(-1))
    ids_w = feat_ids.reshape(-1).astype(jnp.int32)
    emb4, w_flat = _sc_gather(ids4, ids_w, FM_W, FM_V)
    w = w_flat.reshape(B, F)

    a = [params[f"gamma{i}"] * lax.rsqrt(params[f"var{i}"] + 1e-3)
         for i in range(3)]
    c = [params[f"beta{i}"] - params[f"mean{i}"] * a[i] for i in range(3)]
    W0, b0 = params["W0"], params["b0"]
    W1 = a[0][:, None] * params["W1"]
    b1 = c[0] @ params["W1"] + params["b1"]
    W2 = a[1][:, None] * params["W2"]
    b2 = c[1] @ params["W2"] + params["b2"]
    Wo = a[2][:, None] * params["W_out"]
    bfin = c[2] @ params["W_out"] + params["b_out"] + FM_B

    S = np.zeros((F, FD), np.float32)
    S[:, : F * D] = np.repeat(np.eye(F, dtype=np.float32), D, axis=1)
    T = np.zeros((FD, D), np.float32)
    T[: F * D] = np.tile(np.eye(D, dtype=np.float32), (F, 1))
    W0p = jnp.pad(W0, ((0, FD - F * D), (0, 0)))

    pred = _tc_mlp(
        emb4, feat_vals, w, jnp.asarray(S), jnp.asarray(T),
        W0p.astype(jnp.bfloat16), b0.reshape(1, -1),
        W1.astype(jnp.bfloat16), b1.reshape(1, -1),
        W2.astype(jnp.bfloat16), b2.reshape(1, -1), Wo, bfin.reshape(1, 1),
    )
    return pred.reshape(-1)


# speedup vs baseline: 1.6636x; 1.0441x over previous
import functools

import jax
import jax.numpy as jnp
import numpy as np
from jax import lax
from jax.experimental import pallas as pl
from jax.experimental.pallas import tpu as pltpu
from jax.experimental.pallas import tpu_sc as plsc

B = 16384
F = 26
FP = 32
D = 16
BF = B * F
BFP = B * FP

NC = 2
NS = 16
NW = NC * NS
PER_W = BF // NW
PER_W4 = BFP // NW
CH4 = 1024
NCHUNK = PER_W4 // CH4


def _sc_gather(ids4, ids_w, fmw, fmv):
    mesh = plsc.VectorSubcoreMesh(core_axis_name="c", subcore_axis_name="s")

    @functools.partial(
        pl.kernel,
        mesh=mesh,
        compiler_params=pltpu.CompilerParams(use_tc_tiling_on_sc=False),
        out_type=(
            jax.ShapeDtypeStruct((BFP * D,), jnp.float32),
            jax.ShapeDtypeStruct((BF,), jnp.float32),
        ),
        scratch_types=[
            pltpu.VMEM((PER_W4,), jnp.int32),
            pltpu.VMEM((PER_W,), jnp.int32),
            pltpu.VMEM((CH4, D), jnp.float32),
            pltpu.VMEM((CH4, D), jnp.float32),
            pltpu.VMEM((CH4 * D,), jnp.float32),
            pltpu.VMEM((CH4 * D,), jnp.float32),
            pltpu.VMEM((PER_W,), jnp.float32),
            pltpu.SemaphoreType.DMA,
            pltpu.SemaphoreType.DMA,
            pltpu.SemaphoreType.DMA,
            pltpu.SemaphoreType.DMA,
            pltpu.SemaphoreType.DMA,
        ],
    )
    def gk(ids4_hbm, idsw_hbm, fmw_hbm, fmv_hbm, emb_hbm, w_hbm,
           idx4_v, idxw_v, rows_v0, rows_v1, out_v0, out_v1, w_v,
           semg0, semg1, semo0, semo1, sem_w):
        wid = lax.axis_index("s") * NC + lax.axis_index("c")
        base4 = wid * PER_W4
        base = wid * PER_W
        celt = CH4 * D
        rows = [rows_v0, rows_v1]
        outs = [out_v0, out_v1]
        semg = [semg0, semg1]
        semo = [semo0, semo1]
        pltpu.sync_copy(ids4_hbm.at[pl.ds(base4, PER_W4)], idx4_v)
        pltpu.sync_copy(idsw_hbm.at[pl.ds(base, PER_W)], idxw_v)
        wcopy = pltpu.async_copy(fmw_hbm.at[idxw_v], w_v, sem_w)

        def gstart(c):
            b = c % 2
            return pltpu.async_copy(
                fmv_hbm.at[idx4_v.at[pl.ds(c * CH4, CH4)]], rows[b], semg[b])

        gcopies = {0: gstart(0)}
        ocopies = {}
        for c in range(NCHUNK):
            b = c % 2
            if c + 1 < NCHUNK:
                gcopies[c + 1] = gstart(c + 1)
            gcopies[c].wait()
            if c >= 2:
                ocopies[c - 2].wait()

            def repack(m, _):
                for k in range(8):
                    outs[b][pl.ds(m * 128 + k * D, D)] = rows[b][m * 8 + k]
                return 0

            lax.fori_loop(0, CH4 // 8, repack, 0, unroll=4)
            ocopies[c] = pltpu.async_copy(
                outs[b], emb_hbm.at[pl.ds(base4 * D + c * celt, celt)], semo[b])
        ocopies[NCHUNK - 2].wait()
        ocopies[NCHUNK - 1].wait()
        wcopy.wait()
        pltpu.sync_copy(w_v, w_hbm.at[pl.ds(base, PER_W)])

    return gk(ids4, ids_w, fmw, fmv)


BLK = 1024
GRID = B // BLK
FD = FP * D


NB = BLK * FP * D


def _tc_mlp(emb4, vals, w, S, T, W0, b0, W1, b1, W2, b2, Wo, bfin):
    def mk(emb_any, vals_r, w_r, S_r, T_r, W0r, b0r, W1r, b1r, W2r, b2r, Wor,
           bfr, out_r, vbuf, sems):
        i = pl.program_id(0)
        r = lax.rem(i, 2)
        nxt = lax.rem(i + 1, 2)

        @pl.when(i == 0)
        def _():
            pltpu.make_async_copy(
                emb_any.at[pl.ds(0, NB)], vbuf.at[0], sems.at[0]).start()

        @pl.when(i + 1 < GRID)
        def _():
            pltpu.make_async_copy(
                emb_any.at[pl.ds((i + 1) * NB, NB)], vbuf.at[nxt],
                sems.at[nxt]).start()

        pltpu.make_async_copy(
            emb_any.at[pl.ds(i * NB, NB)], vbuf.at[r], sems.at[r]).wait()

        vals_b = vals_r[...]
        y_w = jnp.sum(w_r[...] * vals_b, axis=1, keepdims=True)
        vr = jnp.dot(vals_b, S_r[...], preferred_element_type=jnp.float32,
                     precision=lax.Precision.HIGHEST)
        blk = jnp.reshape(vbuf[r], (BLK // 8, 4, 8, 128))
        parts = [jnp.reshape(blk[:, j], (BLK, 128)) for j in range(4)]
        x = jnp.concatenate(parts, axis=1) * vr
        sv = jnp.dot(x, T_r[...], preferred_element_type=jnp.float32,
                     precision=lax.Precision.HIGHEST)
        y_v = 0.5 * (jnp.sum(sv * sv, axis=1, keepdims=True)
                     - jnp.sum(x * x, axis=1, keepdims=True))
        xb = x.astype(jnp.bfloat16)
        h = jnp.maximum(jnp.dot(xb, W0r[...],
                                preferred_element_type=jnp.float32) + b0r[...], 0.0)
        h = jnp.maximum(jnp.dot(h.astype(jnp.bfloat16), W1r[...],
                                preferred_element_type=jnp.float32) + b1r[...], 0.0)
        h = jnp.maximum(jnp.dot(h.astype(jnp.bfloat16), W2r[...],
                                preferred_element_type=jnp.float32) + b2r[...], 0.0)
        y_d = jnp.dot(h, Wor[...], preferred_element_type=jnp.float32)
        y = y_w + y_v + y_d + bfr[...]
        out_r[...] = jax.nn.sigmoid(y)

    full = lambda a: pl.BlockSpec(a.shape, lambda i: (0,) * a.ndim)
    return pl.pallas_call(
        mk,
        grid=(GRID,),
        in_specs=[
            pl.BlockSpec(memory_space=pl.ANY),
            pl.BlockSpec((BLK, F), lambda i: (i, 0)),
            pl.BlockSpec((BLK, F), lambda i: (i, 0)),
            full(S), full(T), full(W0), full(b0), full(W1), full(b1),
            full(W2), full(b2), full(Wo), full(bfin),
        ],
        out_specs=pl.BlockSpec((BLK, 1), lambda i: (i, 0)),
        out_shape=jax.ShapeDtypeStruct((B, 1), jnp.float32),
        scratch_shapes=[
            pltpu.VMEM((2, NB), jnp.float32),
            pltpu.SemaphoreType.DMA((2,)),
        ],
    )(emb4, vals, w, S, T, W0, b0, W1, b1, W2, b2, Wo, bfin)


def kernel(feat_ids, feat_vals, FM_B, FM_W, FM_V, params):
    pad_ids = jnp.arange(B * (FP - F), dtype=jnp.int32).reshape(B, FP - F)
    ids32 = jnp.concatenate([feat_ids.astype(jnp.int32), pad_ids], axis=1)
    ids4 = (ids32.reshape(B // 8, 8, 4, 8).transpose(0, 2, 1, 3).reshape(-1))
    ids_w = feat_ids.reshape(-1).astype(jnp.int32)
    emb4, w_flat = _sc_gather(ids4, ids_w, FM_W, FM_V)
    w = w_flat.reshape(B, F)

    a = [params[f"gamma{i}"] * lax.rsqrt(params[f"var{i}"] + 1e-3)
         for i in range(3)]
    c = [params[f"beta{i}"] - params[f"mean{i}"] * a[i] for i in range(3)]
    W0, b0 = params["W0"], params["b0"]
    W1 = a[0][:, None] * params["W1"]
    b1 = c[0] @ params["W1"] + params["b1"]
    W2 = a[1][:, None] * params["W2"]
    b2 = c[1] @ params["W2"] + params["b2"]
    Wo = a[2][:, None] * params["W_out"]
    bfin = c[2] @ params["W_out"] + params["b_out"] + FM_B

    S = np.zeros((F, FD), np.float32)
    S[:, : F * D] = np.repeat(np.eye(F, dtype=np.float32), D, axis=1)
    T = np.zeros((FD, D), np.float32)
    T[: F * D] = np.tile(np.eye(D, dtype=np.float32), (F, 1))
    W0p = jnp.pad(W0, ((0, FD - F * D), (0, 0)))

    pred = _tc_mlp(
        emb4, feat_vals, w, jnp.asarray(S), jnp.asarray(T),
        W0p.astype(jnp.bfloat16), b0.reshape(1, -1),
        W1.astype(jnp.bfloat16), b1.reshape(1, -1),
        W2.astype(jnp.bfloat16), b2.reshape(1, -1), Wo, bfin.reshape(1, 1),
    )
    return pred.reshape(-1)


# speedup vs baseline: 1.7028x; 1.0236x over previous
import functools

import jax
import jax.numpy as jnp
import numpy as np
from jax import lax
from jax.experimental import pallas as pl
from jax.experimental.pallas import tpu as pltpu
from jax.experimental.pallas import tpu_sc as plsc

B = 16384
F = 26
D = 16
BF = B * F

NC = 2
NS = 16
NW = NC * NS
PER_W = BF // NW
CH = 1664
NCHUNK = PER_W // CH


def _sc_gather(ids, fmw, fmv):
    mesh = plsc.VectorSubcoreMesh(core_axis_name="c", subcore_axis_name="s")

    @functools.partial(
        pl.kernel,
        mesh=mesh,
        compiler_params=pltpu.CompilerParams(use_tc_tiling_on_sc=False),
        out_type=(
            jax.ShapeDtypeStruct((BF, D), jnp.float32),
            jax.ShapeDtypeStruct((BF,), jnp.float32),
        ),
        scratch_types=[
            pltpu.VMEM((PER_W,), jnp.int32),
            pltpu.VMEM((CH, D), jnp.float32),
            pltpu.VMEM((CH, D), jnp.float32),
            pltpu.VMEM((PER_W,), jnp.float32),
            pltpu.SemaphoreType.DMA,
            pltpu.SemaphoreType.DMA,
            pltpu.SemaphoreType.DMA,
            pltpu.SemaphoreType.DMA,
            pltpu.SemaphoreType.DMA,
        ],
    )
    def gk(ids_hbm, fmw_hbm, fmv_hbm, emb_hbm, w_hbm,
           idx_v, rows_v0, rows_v1, w_v, semg0, semg1, semo0, semo1, sem_w):
        wid = lax.axis_index("s") * NC + lax.axis_index("c")
        base = wid * PER_W
        rows = [rows_v0, rows_v1]
        semg = [semg0, semg1]
        semo = [semo0, semo1]
        pltpu.sync_copy(ids_hbm.at[pl.ds(base, PER_W)], idx_v)
        wcopy = pltpu.async_copy(fmw_hbm.at[idx_v], w_v, sem_w)

        def gstart(c):
            return pltpu.async_copy(
                fmv_hbm.at[idx_v.at[pl.ds(c * CH, CH)]], rows[c % 2],
                semg[c % 2])

        gcopies = {0: gstart(0)}
        ocopies = {}
        for c in range(NCHUNK):
            b = c % 2
            if c + 1 < NCHUNK:
                gcopies[c + 1] = gstart(c + 1)
            gcopies[c].wait()
            if c >= 2:
                ocopies[c - 2].wait()
            ocopies[c] = pltpu.async_copy(
                rows[b], emb_hbm.at[pl.ds(base + c * CH, CH)], semo[b])
        ocopies[NCHUNK - 2].wait()
        ocopies[NCHUNK - 1].wait()
        wcopy.wait()
        pltpu.sync_copy(w_v, w_hbm.at[pl.ds(base, PER_W)])

    return gk(ids, fmw, fmv)


BLK = 1024
GRID = B // BLK


def _tc_mlp(emb, vals, w, S, T, W0, b0, W1, b1, W2, b2, Wo, bfin):
    def mk(emb_r, vals_r, w_r, S_r, T_r, W0r, b0r, W1r, b1r, W2r, b2r, Wor,
           bfr, out_r):
        vals_b = vals_r[...]
        y_w = jnp.sum(w_r[...] * vals_b, axis=1, keepdims=True)
        vr = jnp.dot(vals_b, S_r[...], preferred_element_type=jnp.float32,
                     precision=lax.Precision.HIGHEST)
        x = emb_r[...] * vr
        sv = jnp.dot(x, T_r[...], preferred_element_type=jnp.float32,
                     precision=lax.Precision.HIGHEST)
        y_v = 0.5 * (jnp.sum(sv * sv, axis=1, keepdims=True)
                     - jnp.sum(x * x, axis=1, keepdims=True))
        xb = x.astype(jnp.bfloat16)
        h = jnp.maximum(jnp.dot(xb, W0r[...],
                                preferred_element_type=jnp.float32) + b0r[...], 0.0)
        h = jnp.maximum(jnp.dot(h.astype(jnp.bfloat16), W1r[...],
                                preferred_element_type=jnp.float32) + b1r[...], 0.0)
        h = jnp.maximum(jnp.dot(h.astype(jnp.bfloat16), W2r[...],
                                preferred_element_type=jnp.float32) + b2r[...], 0.0)
        y_d = jnp.dot(h, Wor[...], preferred_element_type=jnp.float32)
        y = y_w + y_v + y_d + bfr[...]
        out_r[...] = jax.nn.sigmoid(y)

    full = lambda a: pl.BlockSpec(a.shape, lambda i: (0,) * a.ndim)
    return pl.pallas_call(
        mk,
        grid=(GRID,),
        in_specs=[
            pl.BlockSpec((BLK, F * D), lambda i: (i, 0)),
            pl.BlockSpec((BLK, F), lambda i: (i, 0)),
            pl.BlockSpec((BLK, F), lambda i: (i, 0)),
            full(S), full(T), full(W0), full(b0), full(W1), full(b1),
            full(W2), full(b2), full(Wo), full(bfin),
        ],
        out_specs=pl.BlockSpec((BLK, 1), lambda i: (i, 0)),
        out_shape=jax.ShapeDtypeStruct((B, 1), jnp.float32),
    )(emb, vals, w, S, T, W0, b0, W1, b1, W2, b2, Wo, bfin)


def kernel(feat_ids, feat_vals, FM_B, FM_W, FM_V, params):
    ids = feat_ids.reshape(-1).astype(jnp.int32)
    emb_flat, w_flat = _sc_gather(ids, FM_W, FM_V)
    emb = emb_flat.reshape(B, F * D)
    w = w_flat.reshape(B, F)

    a = [params[f"gamma{i}"] * lax.rsqrt(params[f"var{i}"] + 1e-3)
         for i in range(3)]
    c = [params[f"beta{i}"] - params[f"mean{i}"] * a[i] for i in range(3)]
    W0, b0 = params["W0"], params["b0"]
    W1 = a[0][:, None] * params["W1"]
    b1 = c[0] @ params["W1"] + params["b1"]
    W2 = a[1][:, None] * params["W2"]
    b2 = c[1] @ params["W2"] + params["b2"]
    Wo = a[2][:, None] * params["W_out"]
    bfin = c[2] @ params["W_out"] + params["b_out"] + FM_B

    S = jnp.asarray(np.repeat(np.eye(F, dtype=np.float32), D, axis=1))
    T = jnp.asarray(np.tile(np.eye(D, dtype=np.float32), (F, 1)))

    pred = _tc_mlp(
        emb, feat_vals, w, S, T,
        W0.astype(jnp.bfloat16), b0.reshape(1, -1),
        W1.astype(jnp.bfloat16), b1.reshape(1, -1),
        W2.astype(jnp.bfloat16), b2.reshape(1, -1), Wo, bfin.reshape(1, 1),
    )
    return pred.reshape(-1)
